# Initial kernel scaffold; baseline (speedup 1.0000x reference)
import jax
import jax.numpy as jnp
from jax.experimental import pallas as pl


def kernel(x, table):
    raise NotImplementedError("write your pallas kernel here")


# speedup vs baseline: 3.8250x; 3.8250x over previous
import jax
import jax.numpy as jnp
from jax import lax
from jax.experimental import pallas as pl
from jax.experimental.pallas import tpu as pltpu
from jax.experimental.pallas import tpu_sc as plsc

N = 16384
S = 200
D = 64
TR = 256
L = 16
NC, NS = 2, 16
NW = NC * NS
RPW = N // NW
BR = L
NBLK = RPW // BR
CR = 2
NSUB = BR // CR
SUB_IDX = CR * S
SUB_OUT = CR * S * D
BLK_IDX = BR * S


def _body(x_hbm, table_hbm, out_hbm, xbuf, idxbuf, outb0, outb1, tbuf,
          sem0, sem1):
    wid = lax.axis_index("s") * NC + lax.axis_index("c")
    pltpu.sync_copy(table_hbm, tbuf)
    lane = lax.iota(jnp.int32, L)
    onev = jnp.ones((L,), jnp.int32)
    zerov = jnp.zeros((L,), jnp.int32)
    sixteenv = jnp.full((L,), L, jnp.int32)
    d64v = jnp.full((L,), D, jnp.int32)
    col0 = lane * S
    outbufs = (outb0, outb1)
    sems = (sem0, sem1)

    def blk_body(b, _):
        base_row = wid * RPW + b * BR
        pltpu.sync_copy(x_hbm.at[pl.ds(base_row * S, BLK_IDX)], xbuf)

        def pos_body(j, carry):
            idxcur, acc = carry
            col = plsc.load_gather(xbuf, [idxcur])
            m = (col != zerov).astype(jnp.int32)
            acc = acc + m
            plsc.store_scatter(idxbuf, [idxcur], acc * m)
            return idxcur + onev, acc

        lax.fori_loop(0, S, pos_body, (col0, zerov))

        def sub2_body(i, _):
            for hb in range(2):
                s = 2 * i + hb
                gsub = b * NSUB + s
                obuf, sem = outbufs[hb], sems[hb]

                @pl.when(gsub >= 2)
                def _():
                    pltpu.make_async_copy(
                        obuf, out_hbm.at[pl.ds(0, SUB_OUT)], sem).wait()

                def grp_body(g, dstv):
                    off = s * SUB_IDX + g * L
                    idxv = idxbuf[pl.ds(off, L)]
                    for k in range(L):
                        pos = idxv[k]
                        src = pos * D + lane
                        dst = dstv
                        for _m in range(D // L):
                            v = plsc.load_gather(tbuf, [src])
                            plsc.store_scatter(obuf, [dst], v)
                            src = src + sixteenv
                            dst = dst + sixteenv
                        dstv = dstv + d64v
                    return dstv

                lax.fori_loop(0, SUB_IDX // L, grp_body, lane)
                pltpu.async_copy(
                    obuf,
                    out_hbm.at[pl.ds((base_row + s * CR) * S * D, SUB_OUT)],
                    sem)
            return 0

        lax.fori_loop(0, NSUB // 2, sub2_body, 0)
        return 0

    lax.fori_loop(0, NBLK, blk_body, 0)
    for hb in range(2):
        pltpu.make_async_copy(
            outbufs[hb], out_hbm.at[pl.ds(0, SUB_OUT)], sems[hb]).wait()


@jax.jit
def kernel(x, table):
    x_flat = x.reshape(-1).astype(jnp.int32)
    t_flat = table.reshape(-1)
    mesh = plsc.VectorSubcoreMesh(core_axis_name="c", subcore_axis_name="s")
    k = pl.kernel(
        _body,
        mesh=mesh,
        compiler_params=pltpu.CompilerParams(needs_layout_passes=False),
        out_type=jax.ShapeDtypeStruct((N * S * D,), jnp.float32),
        scratch_types=[
            pltpu.VMEM((BLK_IDX,), jnp.int32),
            pltpu.VMEM((BLK_IDX,), jnp.int32),
            pltpu.VMEM((SUB_OUT,), jnp.float32),
            pltpu.VMEM((SUB_OUT,), jnp.float32),
            pltpu.VMEM((TR * D,), jnp.float32),
            pltpu.SemaphoreType.DMA,
            pltpu.SemaphoreType.DMA,
        ],
    )
    out = k(x_flat, t_flat)
    return out.reshape(N, S, D)


# speedup vs baseline: 5.7203x; 1.4955x over previous
import jax
import jax.numpy as jnp
from jax import lax
from jax.experimental import pallas as pl
from jax.experimental.pallas import tpu as pltpu
from jax.experimental.pallas import tpu_sc as plsc

N = 16384
S = 200
D = 64
TR = 256
L = 16
NC, NS = 2, 16
NW = NC * NS
RPW = N // NW
BR = 128
NBLK = RPW // BR
NG = BR // L
BLK_X = BR * S
NTD = D // 8
SLAB = D * BR
ROW_STRIDE = N // BR * BR * 8
S_STRIDE = D * N


def _body(x_hbm, table_hbm, out_hbm, xbuf, idxbuf, stg0, stg1, tbuf, tbufT,
          sem0, sem1):
    wid = lax.axis_index("s") * NC + lax.axis_index("c")
    pltpu.sync_copy(table_hbm, tbuf)
    lane = lax.iota(jnp.int32, L)
    onev = jnp.ones((L,), jnp.int32)
    zerov = jnp.zeros((L,), jnp.int32)
    v128 = jnp.full((L,), BR, jnp.int32)
    v256 = jnp.full((L,), TR, jnp.int32)
    lane64 = lane * D
    lane200 = lane * S
    stgs = (stg0, stg1)
    sems = (sem0, sem1)

    def tp_d(d, _):
        def tp_pg(pg, _):
            v = plsc.load_gather(tbuf, [lane64 + (pg * (L * D) + d)])
            tbufT[pl.ds(d * TR + pg * L, L)] = v
            return 0
        lax.fori_loop(0, TR // L, tp_pg, 0)
        return 0
    lax.fori_loop(0, D, tp_d, 0)

    def blk_body(tbi, _):
        tb = wid * NBLK + tbi
        pltpu.sync_copy(x_hbm.at[pl.ds(tb * BLK_X, BLK_X)], xbuf)

        for g in range(NG):
            def pos_body(j, carry):
                xaddr, paddr, acc = carry
                col = plsc.load_gather(xbuf, [xaddr])
                m = (col != zerov).astype(jnp.int32)
                acc = acc + m
                plsc.store_scatter(idxbuf, [paddr], acc * m)
                return xaddr + onev, paddr + v128, acc
            lax.fori_loop(0, S, pos_body,
                          (lane200 + g * (L * S), lane + g * L, zerov))

        def s2_body(i, _):
            for hb in range(2):
                s = 2 * i + hb
                stg, sem = stgs[hb], sems[hb]

                @pl.when(tbi * S + s >= 2)
                def _():
                    pltpu.make_async_copy(
                        stg, out_hbm.at[pl.ds(0, SLAB)], sem).wait()

                for g in range(NG):
                    posv = idxbuf[pl.ds(s * BR + g * L, L)]
                    addrv = posv
                    for d in range(D):
                        v = plsc.load_gather(tbufT, [addrv])
                        stg[pl.ds((d // 8) * 1024 + (d % 8) * BR + g * L,
                                  L)] = v
                        addrv = addrv + v256

                for td in range(NTD):
                    pltpu.async_copy(
                        stg.at[pl.ds(td * 1024, 1024)],
                        out_hbm.at[pl.ds(
                            s * S_STRIDE + td * ROW_STRIDE + tb * 1024,
                            1024)],
                        sem)
            return 0

        lax.fori_loop(0, S // 2, s2_body, 0)
        return 0

    lax.fori_loop(0, NBLK, blk_body, 0)
    for hb in range(2):
        pltpu.make_async_copy(
            stgs[hb], out_hbm.at[pl.ds(0, SLAB)], sems[hb]).wait()


@jax.jit
def kernel(x, table):
    x_flat = x.reshape(-1).astype(jnp.int32)
    t_flat = table.reshape(-1)
    mesh = plsc.VectorSubcoreMesh(core_axis_name="c", subcore_axis_name="s")
    k = pl.kernel(
        _body,
        mesh=mesh,
        compiler_params=pltpu.CompilerParams(needs_layout_passes=False),
        out_type=jax.ShapeDtypeStruct((N * S * D,), jnp.float32),
        scratch_types=[
            pltpu.VMEM((BLK_X,), jnp.int32),
            pltpu.VMEM((S * BR,), jnp.int32),
            pltpu.VMEM((SLAB,), jnp.float32),
            pltpu.VMEM((SLAB,), jnp.float32),
            pltpu.VMEM((TR * D,), jnp.float32),
            pltpu.VMEM((D * TR,), jnp.float32),
            pltpu.SemaphoreType.DMA,
            pltpu.SemaphoreType.DMA,
        ],
    )
    out = k(x_flat, t_flat)
    f5 = out.reshape(S, D // 8, N // 128, 8, 128)
    return f5.transpose(2, 4, 0, 1, 3).reshape(N, S, D)


# speedup vs baseline: 11.2847x; 1.9728x over previous
import jax
import jax.numpy as jnp
from jax import lax
from jax.experimental import pallas as pl
from jax.experimental.pallas import tpu as pltpu
from jax.experimental.pallas import tpu_sc as plsc

N = 16384
S = 200
D = 64
TR = 256
L = 16
NC, NS = 2, 16
NW = NC * NS
RPW = N // NW
BR = 128
NBLK = RPW // BR
NG = BR // L
BLK_X = BR * S
NTD = D // 8
SLAB = D * BR
ROW_STRIDE = N // BR * BR * 8
S_STRIDE = D * N


def _body(x_hbm, table_hbm, out_hbm, xbuf, idxbuf, stg0, stg1, tbuf, tbufT,
          sem0, sem1):
    wid = lax.axis_index("s") * NC + lax.axis_index("c")
    pltpu.sync_copy(table_hbm, tbuf)
    lane = lax.iota(jnp.int32, L)
    onev = jnp.ones((L,), jnp.int32)
    zerov = jnp.zeros((L,), jnp.int32)
    v128 = jnp.full((L,), BR, jnp.int32)
    v256 = jnp.full((L,), TR, jnp.int32)
    lane64 = lane * D
    lane200 = lane * S
    stgs = (stg0, stg1)
    sems = (sem0, sem1)

    def tp_d(d, _):
        def tp_pg(pg, _):
            v = plsc.load_gather(tbuf, [lane64 + (pg * (L * D) + d)])
            tbufT[pl.ds(d * TR + pg * L, L)] = v
            return 0
        lax.fori_loop(0, TR // L, tp_pg, 0)
        return 0
    lax.fori_loop(0, D, tp_d, 0)

    def blk_body(tbi, _):
        tb = wid * NBLK + tbi
        pltpu.sync_copy(x_hbm.at[pl.ds(tb * BLK_X, BLK_X)], xbuf)

        for g in range(NG):
            def pos_body(j, carry):
                xaddr, paddr, acc = carry
                col = plsc.load_gather(xbuf, [xaddr])
                m = (col != zerov).astype(jnp.int32)
                acc = acc + m
                plsc.store_scatter(idxbuf, [paddr], acc * m)
                return xaddr + onev, paddr + v128, acc
            lax.fori_loop(0, S, pos_body,
                          (lane200 + g * (L * S), lane + g * L, zerov))

        def s2_body(i, _):
            for hb in range(2):
                s = 2 * i + hb
                stg, sem = stgs[hb], sems[hb]

                @pl.when(tbi * S + s >= 2)
                def _():
                    pltpu.make_async_copy(
                        stg, out_hbm.at[0, :, 0], sem).wait()

                posvs = [idxbuf[pl.ds(s * BR + g * L, L)]
                         for g in range(NG)]
                pos0 = posvs[0][0]
                eqv = posvs[0] == pos0
                for g in range(1, NG):
                    eqv = jnp.logical_and(eqv, posvs[g] == pos0)
                alleq = jnp.all(eqv)

                @pl.when(alleq)
                def _():
                    rvs = [tbuf[pl.ds(pos0 * D + j * L, L)]
                           for j in range(D // L)]
                    for d in range(D):
                        bv = jnp.broadcast_to(rvs[d // L][d % L], (L,))
                        base = (d % 8) * BR
                        for g in range(NG):
                            stg[d // 8, pl.ds(base + g * L, L)] = bv

                @pl.when(jnp.logical_not(alleq))
                def _():
                    for g in range(NG):
                        addrv = posvs[g]
                        for td in range(NTD):
                            vs = []
                            for _dd in range(8):
                                vs.append(plsc.load_gather(tbufT, [addrv]))
                                addrv = addrv + v256
                            for dd in range(8):
                                stg[td, pl.ds(dd * BR + g * L, L)] = vs[dd]

                pltpu.async_copy(stg, out_hbm.at[s, :, tb], sem)
            return 0

        lax.fori_loop(0, S // 2, s2_body, 0)
        return 0

    lax.fori_loop(0, NBLK, blk_body, 0)
    for hb in range(2):
        pltpu.make_async_copy(
            stgs[hb], out_hbm.at[0, :, 0], sems[hb]).wait()


@jax.jit
def kernel(x, table):
    x_flat = x.reshape(-1).astype(jnp.int32)
    t_flat = table.reshape(-1)
    mesh = plsc.VectorSubcoreMesh(core_axis_name="c", subcore_axis_name="s")
    k = pl.kernel(
        _body,
        mesh=mesh,
        compiler_params=pltpu.CompilerParams(needs_layout_passes=False),
        out_type=jax.ShapeDtypeStruct((S, NTD, N // BR, 8 * BR),
                                      jnp.float32),
        scratch_types=[
            pltpu.VMEM((BLK_X,), jnp.int32),
            pltpu.VMEM((S * BR,), jnp.int32),
            pltpu.VMEM((NTD, 8 * BR), jnp.float32),
            pltpu.VMEM((NTD, 8 * BR), jnp.float32),
            pltpu.VMEM((TR * D,), jnp.float32),
            pltpu.VMEM((D * TR,), jnp.float32),
            pltpu.SemaphoreType.DMA,
            pltpu.SemaphoreType.DMA,
        ],
    )
    out = k(x_flat, t_flat)
    f5 = out.reshape(S, D // 8, N // 128, 8, 128)
    return f5.transpose(2, 4, 0, 1, 3).reshape(N, S, D)


# speedup vs baseline: 21.9954x; 1.9491x over previous
import jax
import jax.numpy as jnp
from jax import lax
from jax.experimental import pallas as pl
from jax.experimental.pallas import tpu as pltpu
from jax.experimental.pallas import tpu_sc as plsc

N = 16384
S = 200
D = 64
TR = 256
L = 16
NC, NS = 2, 16
NW = NC * NS
RPW = N // NW
BR = 128
NBLK = RPW // BR
NG = BR // L
BLK_X = BR * S
NTD = D // 8
SLAB = D * BR
ROW_STRIDE = N // BR * BR * 8
S_STRIDE = D * N


def _body(x_hbm, table_hbm, out_hbm, xbuf, idxbuf, stg0, stg1, tbuf, tbufT,
          sem0, sem1):
    wid = lax.axis_index("s") * NC + lax.axis_index("c")
    pltpu.sync_copy(table_hbm, tbuf)
    lane = lax.iota(jnp.int32, L)
    onev = jnp.ones((L,), jnp.int32)
    zerov = jnp.zeros((L,), jnp.int32)
    v128 = jnp.full((L,), BR, jnp.int32)
    v256 = jnp.full((L,), TR, jnp.int32)
    lane64 = lane * D
    lane200 = lane * S
    stgs = (stg0, stg1)
    sems = (sem0, sem1)

    def tp_d(d, _):
        def tp_pg(pg, _):
            v = plsc.load_gather(tbuf, [lane64 + (pg * (L * D) + d)])
            tbufT[pl.ds(d * TR + pg * L, L)] = v
            return 0
        lax.fori_loop(0, TR // L, tp_pg, 0)
        return 0
    lax.fori_loop(0, D, tp_d, 0)

    def blk_body(tbi, _):
        tb = wid * NBLK + tbi
        pltpu.sync_copy(x_hbm.at[pl.ds(tb * BLK_X, BLK_X)], xbuf)

        for g in range(NG):
            def pos_body(j, carry):
                xaddr, paddr, acc = carry
                col = plsc.load_gather(xbuf, [xaddr])
                m = (col != zerov).astype(jnp.int32)
                acc = acc + m
                plsc.store_scatter(idxbuf, [paddr], acc * m)
                return xaddr + onev, paddr + v128, acc
            lax.fori_loop(0, S, pos_body,
                          (lane200 + g * (L * S), lane + g * L, zerov))

        def s2_body(i, _):
            for hb in range(2):
                s = 2 * i + hb
                stg, sem = stgs[hb], sems[hb]

                @pl.when(tbi * S + s >= 2)
                def _():
                    pltpu.make_async_copy(
                        stg, out_hbm.at[pl.ds(0, SLAB)], sem).wait()

                posvs = [idxbuf[pl.ds(s * BR + g * L, L)]
                         for g in range(NG)]
                pos0 = posvs[0][0]
                eqv = posvs[0] == pos0
                for g in range(1, NG):
                    eqv = jnp.logical_and(eqv, posvs[g] == pos0)
                alleq = jnp.all(eqv)

                @pl.when(alleq)
                def _():
                    rvs = [tbuf[pl.ds(pos0 * D + j * L, L)]
                           for j in range(D // L)]
                    for d in range(D):
                        bv = jnp.broadcast_to(rvs[d // L][d % L], (L,))
                        base = (d // 8) * 1024 + (d % 8) * BR
                        for g in range(NG):
                            stg[pl.ds(base + g * L, L)] = bv

                @pl.when(jnp.logical_not(alleq))
                def _():
                    for g in range(NG):
                        addrv = posvs[g]
                        for td in range(NTD):
                            vs = []
                            for _dd in range(8):
                                vs.append(plsc.load_gather(tbufT, [addrv]))
                                addrv = addrv + v256
                            for dd in range(8):
                                stg[pl.ds(td * 1024 + dd * BR + g * L,
                                          L)] = vs[dd]

                for td in range(NTD):
                    pltpu.async_copy(
                        stg.at[pl.ds(td * 1024, 1024)],
                        out_hbm.at[pl.ds(
                            s * S_STRIDE + td * ROW_STRIDE + tb * 1024,
                            1024)],
                        sem)
            return 0

        lax.fori_loop(0, S // 2, s2_body, 0)
        return 0

    lax.fori_loop(0, NBLK, blk_body, 0)
    for hb in range(2):
        pltpu.make_async_copy(
            stgs[hb], out_hbm.at[pl.ds(0, SLAB)], sems[hb]).wait()


@jax.jit
def kernel(x, table):
    x_flat = x.reshape(-1).astype(jnp.int32)
    t_flat = table.reshape(-1)
    mesh = plsc.VectorSubcoreMesh(core_axis_name="c", subcore_axis_name="s")
    k = pl.kernel(
        _body,
        mesh=mesh,
        compiler_params=pltpu.CompilerParams(needs_layout_passes=False),
        out_type=jax.ShapeDtypeStruct((N * S * D,), jnp.float32),
        scratch_types=[
            pltpu.VMEM((BLK_X,), jnp.int32),
            pltpu.VMEM((S * BR,), jnp.int32),
            pltpu.VMEM((SLAB,), jnp.float32),
            pltpu.VMEM((SLAB,), jnp.float32),
            pltpu.VMEM((TR * D,), jnp.float32),
            pltpu.VMEM((D * TR,), jnp.float32),
            pltpu.SemaphoreType.DMA,
            pltpu.SemaphoreType.DMA,
        ],
    )
    out = k(x_flat, t_flat)
    f5 = out.reshape(S, D // 8, N // 128, 8, 128)
    return f5.transpose(2, 4, 0, 1, 3).reshape(N, S, D)
